# initial kernel scaffold (unmeasured)
import jax
import jax.numpy as jnp
from jax import lax
from jax.experimental import pallas as pl
from jax.experimental.pallas import tpu as pltpu


def kernel(
    x,
):
    def body(*refs):
        pass

    out_shape = jax.ShapeDtypeStruct(..., jnp.float32)
    return pl.pallas_call(body, out_shape=out_shape)(...)


# baseline (device time: 430355 ns/iter reference)
import jax
import jax.numpy as jnp
from jax import lax
from jax.experimental import pallas as pl
from jax.experimental.pallas import tpu as pltpu

_CAST_BLOCKS = 16
_ADD_BLOCKS = 16


def kernel(x):
    m, n = x.shape

    bm = m // _CAST_BLOCKS

    def cast_body(x_ref, o_ref):
        o_ref[...] = x_ref[...].astype(jnp.bfloat16)

    y = pl.pallas_call(
        cast_body,
        grid=(_CAST_BLOCKS,),
        in_specs=[pl.BlockSpec((bm, n), lambda i: (i, 0))],
        out_specs=pl.BlockSpec((bm, n), lambda i: (i, 0)),
        out_shape=jax.ShapeDtypeStruct((m, n), jnp.bfloat16),
    )(x)

    def exch_body(y_ref, recv_ref, send_sem, recv_sem):
        my_x = lax.axis_index("x")
        my_y = lax.axis_index("y")
        my_z = lax.axis_index("z")
        partner = (my_x, 1 - my_y, my_z)

        barrier_sem = pltpu.get_barrier_semaphore()
        pl.semaphore_signal(
            barrier_sem, inc=1, device_id=partner,
            device_id_type=pl.DeviceIdType.MESH,
        )
        pl.semaphore_wait(barrier_sem, 1)

        rdma = pltpu.make_async_remote_copy(
            src_ref=y_ref,
            dst_ref=recv_ref,
            send_sem=send_sem,
            recv_sem=recv_sem,
            device_id=partner,
            device_id_type=pl.DeviceIdType.MESH,
        )
        rdma.start()
        rdma.wait()

    recv = pl.pallas_call(
        exch_body,
        out_shape=jax.ShapeDtypeStruct((m, n), jnp.bfloat16),
        in_specs=[pl.BlockSpec(memory_space=pl.ANY)],
        out_specs=pl.BlockSpec(memory_space=pl.ANY),
        scratch_shapes=[
            pltpu.SemaphoreType.DMA,
            pltpu.SemaphoreType.DMA,
        ],
        compiler_params=pltpu.CompilerParams(collective_id=0),
    )(y)

    am = m // _ADD_BLOCKS

    def add_body(a_ref, b_ref, o_ref):
        o_ref[...] = a_ref[...] + b_ref[...]

    return pl.pallas_call(
        add_body,
        grid=(_ADD_BLOCKS,),
        in_specs=[
            pl.BlockSpec((am, n), lambda i: (i, 0)),
            pl.BlockSpec((am, n), lambda i: (i, 0)),
        ],
        out_specs=pl.BlockSpec((am, n), lambda i: (i, 0)),
        out_shape=jax.ShapeDtypeStruct((m, n), jnp.bfloat16),
    )(y, recv)


# device time: 278338 ns/iter; 1.5462x vs baseline; 1.5462x over previous
import jax
import jax.numpy as jnp
from jax import lax
from jax.experimental import pallas as pl
from jax.experimental.pallas import tpu as pltpu

_CAST_BLOCKS = 16
_K = 8


def kernel(x):
    m, n = x.shape
    half = m // 2
    ch = half // _K

    bm = m // _CAST_BLOCKS

    def cast_body(x_ref, o_ref):
        o_ref[...] = x_ref[...].astype(jnp.bfloat16)

    y = pl.pallas_call(
        cast_body,
        grid=(_CAST_BLOCKS,),
        in_specs=[pl.BlockSpec((bm, n), lambda i: (i, 0))],
        out_specs=pl.BlockSpec((bm, n), lambda i: (i, 0)),
        out_shape=jax.ShapeDtypeStruct((m, n), jnp.bfloat16),
    )(x)

    def body(
        y_ref,
        out_ref,
        recv_ref,
        y_send, y_recv, x_send, x_recv,
        b_in, o_out,
        b_slots, o_slots,
    ):
        my_x = lax.axis_index("x")
        my_y = lax.axis_index("y")
        my_z = lax.axis_index("z")
        partner = (my_x, 1 - my_y, my_z)
        xnbr = (1 - my_x, my_y, my_z)

        barrier_sem = pltpu.get_barrier_semaphore()
        for nbr in (partner, xnbr):
            pl.semaphore_signal(
                barrier_sem, inc=1, device_id=nbr,
                device_id_type=pl.DeviceIdType.MESH,
            )
        pl.semaphore_wait(barrier_sem, 2)

        my_half = my_x * half
        other_half = (1 - my_x) * half

        y_rdmas = []
        for k in range(_K):
            sl = pl.ds(my_half + k * ch, ch)
            r = pltpu.make_async_remote_copy(
                src_ref=y_ref.at[sl, :],
                dst_ref=recv_ref.at[sl, :],
                send_sem=y_send.at[k],
                recv_sem=y_recv.at[k],
                device_id=partner,
                device_id_type=pl.DeviceIdType.MESH,
            )
            r.start()
            y_rdmas.append(r)

        o_cps = []

        def do_add(c_start, idx):
            slot = idx % 2
            if idx >= 2:
                o_cps[idx - 2].wait()
            sl = pl.ds(c_start, ch)
            b_cp = pltpu.make_async_copy(
                recv_ref.at[sl, :], b_slots.at[slot], b_in.at[slot]
            )
            b_cp.start()
            b_cp.wait()
            o_slots[slot, :, :] = y_ref[sl, :] + b_slots[slot, :, :]
            o_cp = pltpu.make_async_copy(
                o_slots.at[slot], out_ref.at[sl, :], o_out.at[slot]
            )
            o_cp.start()
            o_cps.append(o_cp)

        x_rdmas = []
        for k in range(_K):
            c_start = my_half + k * ch
            sl = pl.ds(c_start, ch)
            y_rdmas[k].wait_recv()
            fwd = pltpu.make_async_remote_copy(
                src_ref=recv_ref.at[sl, :],
                dst_ref=recv_ref.at[sl, :],
                send_sem=x_send.at[k],
                recv_sem=x_recv.at[k],
                device_id=xnbr,
                device_id_type=pl.DeviceIdType.MESH,
            )
            fwd.start()
            x_rdmas.append(fwd)
            do_add(c_start, k)

        for k in range(_K):
            c_start = other_half + k * ch
            sl = pl.ds(c_start, ch)
            rx = pltpu.make_async_remote_copy(
                src_ref=recv_ref.at[sl, :],
                dst_ref=recv_ref.at[sl, :],
                send_sem=y_send.at[k],
                recv_sem=x_recv.at[k],
                device_id=xnbr,
                device_id_type=pl.DeviceIdType.MESH,
            )
            rx.wait_recv()
            do_add(c_start, _K + k)

        o_cps[-2].wait()
        o_cps[-1].wait()
        for k in range(_K):
            y_rdmas[k].wait_send()
            x_rdmas[k].wait_send()

    out, _recv = pl.pallas_call(
        body,
        out_shape=[
            jax.ShapeDtypeStruct((m, n), jnp.bfloat16),
            jax.ShapeDtypeStruct((m, n), jnp.bfloat16),
        ],
        in_specs=[pl.BlockSpec(memory_space=pltpu.MemorySpace.VMEM)],
        out_specs=[
            pl.BlockSpec(memory_space=pl.ANY),
            pl.BlockSpec(memory_space=pl.ANY),
        ],
        scratch_shapes=[
            pltpu.SemaphoreType.DMA((_K,)),
            pltpu.SemaphoreType.DMA((_K,)),
            pltpu.SemaphoreType.DMA((_K,)),
            pltpu.SemaphoreType.DMA((_K,)),
            pltpu.SemaphoreType.DMA((2,)),
            pltpu.SemaphoreType.DMA((2,)),
            pltpu.VMEM((2, ch, n), jnp.bfloat16),
            pltpu.VMEM((2, ch, n), jnp.bfloat16),
        ],
        compiler_params=pltpu.CompilerParams(
            collective_id=0,
            vmem_limit_bytes=56 * 1024 * 1024,
        ),
    )(y)
    return out


# device time: 245174 ns/iter; 1.7553x vs baseline; 1.1353x over previous
import jax
import jax.numpy as jnp
from jax import lax
from jax.experimental import pallas as pl
from jax.experimental.pallas import tpu as pltpu

_K = 16


def kernel(x):
    m, n = x.shape
    half = m // 2
    ch = half // _K

    def body(
        x_hbm,
        out_ref,
        y_half,
        recv_y,
        recv_x,
        xin,
        o_slots,
        y_send, y_recv, x_send, x_recv,
        xin_sem, o_out,
    ):
        my_x = lax.axis_index("x")
        my_y = lax.axis_index("y")
        my_z = lax.axis_index("z")
        partner = (my_x, 1 - my_y, my_z)
        xnbr = (1 - my_x, my_y, my_z)

        barrier_sem = pltpu.get_barrier_semaphore()
        for nbr in (partner, xnbr):
            pl.semaphore_signal(
                barrier_sem, inc=1, device_id=nbr,
                device_id_type=pl.DeviceIdType.MESH,
            )
        pl.semaphore_wait(barrier_sem, 2)

        my_half = my_x * half
        other_half = (1 - my_x) * half

        cp = pltpu.make_async_copy(
            x_hbm.at[pl.ds(my_half, ch), :], xin.at[0], xin_sem.at[0]
        )
        cp.start()
        prev = cp
        y_rdmas = []
        for k in range(_K):
            slot = k % 2
            prev.wait()
            if k + 1 < _K:
                nxt = pltpu.make_async_copy(
                    x_hbm.at[pl.ds(my_half + (k + 1) * ch, ch), :],
                    xin.at[(k + 1) % 2],
                    xin_sem.at[(k + 1) % 2],
                )
                nxt.start()
                prev = nxt
            y_half[k * ch:(k + 1) * ch, :] = xin[slot].astype(jnp.bfloat16)
            r = pltpu.make_async_remote_copy(
                src_ref=y_half.at[pl.ds(k * ch, ch), :],
                dst_ref=recv_y.at[pl.ds(k * ch, ch), :],
                send_sem=y_send.at[k],
                recv_sem=y_recv.at[k],
                device_id=partner,
                device_id_type=pl.DeviceIdType.MESH,
            )
            r.start()
            y_rdmas.append(r)

        o_cps = []

        def emit(o_chunk_start, slot):
            ocp = pltpu.make_async_copy(
                o_slots.at[slot],
                out_ref.at[pl.ds(o_chunk_start, ch), :],
                o_out.at[slot],
            )
            ocp.start()
            o_cps.append(ocp)

        x_rdmas = []
        for k in range(_K):
            sl = pl.ds(k * ch, ch)
            y_rdmas[k].wait_recv()
            fwd = pltpu.make_async_remote_copy(
                src_ref=recv_y.at[sl, :],
                dst_ref=recv_x.at[sl, :],
                send_sem=x_send.at[k],
                recv_sem=x_recv.at[k],
                device_id=xnbr,
                device_id_type=pl.DeviceIdType.MESH,
            )
            fwd.start()
            x_rdmas.append(fwd)
            slot = k % 2
            if len(o_cps) >= 2:
                o_cps[-2].wait()
            o_slots[slot, :, :] = y_half[sl, :] + recv_y[sl, :]
            emit(my_half + k * ch, slot)

        for k in range(_K):
            slot = k % 2
            sl = pl.ds(k * ch, ch)
            gsl = pl.ds(other_half + k * ch, ch)
            xcp = pltpu.make_async_copy(
                x_hbm.at[gsl, :], xin.at[slot], xin_sem.at[slot]
            )
            xcp.start()
            rx = pltpu.make_async_remote_copy(
                src_ref=recv_x.at[sl, :],
                dst_ref=recv_x.at[sl, :],
                send_sem=x_send.at[k],
                recv_sem=x_recv.at[k],
                device_id=xnbr,
                device_id_type=pl.DeviceIdType.MESH,
            )
            rx.wait_recv()
            xcp.wait()
            if len(o_cps) >= 2:
                o_cps[-2].wait()
            o_slots[slot, :, :] = (
                xin[slot].astype(jnp.bfloat16) + recv_x[sl, :]
            )
            emit(other_half + k * ch, slot)

        o_cps[-2].wait()
        o_cps[-1].wait()
        for k in range(_K):
            y_rdmas[k].wait_send()
            x_rdmas[k].wait_send()

    return pl.pallas_call(
        body,
        out_shape=jax.ShapeDtypeStruct((m, n), jnp.bfloat16),
        in_specs=[pl.BlockSpec(memory_space=pl.ANY)],
        out_specs=pl.BlockSpec(memory_space=pl.ANY),
        scratch_shapes=[
            pltpu.VMEM((half, n), jnp.bfloat16),
            pltpu.VMEM((half, n), jnp.bfloat16),
            pltpu.VMEM((half, n), jnp.bfloat16),
            pltpu.VMEM((2, ch, n), jnp.float32),
            pltpu.VMEM((2, ch, n), jnp.bfloat16),
            pltpu.SemaphoreType.DMA((_K,)),
            pltpu.SemaphoreType.DMA((_K,)),
            pltpu.SemaphoreType.DMA((_K,)),
            pltpu.SemaphoreType.DMA((_K,)),
            pltpu.SemaphoreType.DMA((2,)),
            pltpu.SemaphoreType.DMA((2,)),
        ],
        compiler_params=pltpu.CompilerParams(
            collective_id=0,
            vmem_limit_bytes=58 * 1024 * 1024,
        ),
    )(x)


# device time: 233462 ns/iter; 1.8434x vs baseline; 1.0502x over previous
import jax
import jax.numpy as jnp
from jax import lax
from jax.experimental import pallas as pl
from jax.experimental.pallas import tpu as pltpu

_K = 16


def kernel(x):
    m, n = x.shape
    half = m // 2
    ch = half // _K

    def body(
        x_hbm,
        out_ref,
        y_half,
        recv_y,
        recv_x,
        xin,
        o_slots,
        y_send, y_recv, x_send, x_recv,
        xin_sem, o_out,
    ):
        my_x = lax.axis_index("x")
        my_y = lax.axis_index("y")
        my_z = lax.axis_index("z")
        partner = (my_x, 1 - my_y, my_z)
        xnbr = (1 - my_x, my_y, my_z)

        barrier_sem = pltpu.get_barrier_semaphore()
        for nbr in (partner, xnbr):
            pl.semaphore_signal(
                barrier_sem, inc=1, device_id=nbr,
                device_id_type=pl.DeviceIdType.MESH,
            )
        pl.semaphore_wait(barrier_sem, 2)

        my_half = my_x * half
        other_half = (1 - my_x) * half

        cp = pltpu.make_async_copy(
            x_hbm.at[pl.ds(my_half, ch), :], xin.at[0], xin_sem.at[0]
        )
        cp.start()
        prev = cp
        y_rdmas = []
        for k in range(_K):
            slot = k % 2
            prev.wait()
            if k + 1 < _K:
                nxt = pltpu.make_async_copy(
                    x_hbm.at[pl.ds(my_half + (k + 1) * ch, ch), :],
                    xin.at[(k + 1) % 2],
                    xin_sem.at[(k + 1) % 2],
                )
                nxt.start()
                prev = nxt
            y_half[k * ch:(k + 1) * ch, :] = xin[slot].astype(jnp.bfloat16)
            r = pltpu.make_async_remote_copy(
                src_ref=y_half.at[pl.ds(k * ch, ch), :],
                dst_ref=recv_y.at[pl.ds(k * ch, ch), :],
                send_sem=y_send.at[k],
                recv_sem=y_recv.at[k],
                device_id=partner,
                device_id_type=pl.DeviceIdType.MESH,
            )
            r.start()
            y_rdmas.append(r)

        o_cps = []

        def emit(o_chunk_start, slot):
            ocp = pltpu.make_async_copy(
                o_slots.at[slot],
                out_ref.at[pl.ds(o_chunk_start, ch), :],
                o_out.at[slot],
            )
            ocp.start()
            o_cps.append(ocp)

        x_rdmas = []
        xin_cps = [None] * _K
        o_emitted = [0]

        def stage_xin(j):
            if j < _K:
                xcp = pltpu.make_async_copy(
                    x_hbm.at[pl.ds(other_half + j * ch, ch), :],
                    xin.at[j % 2],
                    xin_sem.at[j % 2],
                )
                xcp.start()
                xin_cps[j] = xcp

        def process_y(k):
            sl = pl.ds(k * ch, ch)
            y_rdmas[k].wait_recv()
            fwd = pltpu.make_async_remote_copy(
                src_ref=recv_y.at[sl, :],
                dst_ref=recv_x.at[sl, :],
                send_sem=x_send.at[k],
                recv_sem=x_recv.at[k],
                device_id=xnbr,
                device_id_type=pl.DeviceIdType.MESH,
            )
            fwd.start()
            x_rdmas.append(fwd)
            slot = o_emitted[0] % 2
            if len(o_cps) >= 2:
                o_cps[-2].wait()
            o_slots[slot, :, :] = y_half[sl, :] + recv_y[sl, :]
            emit(my_half + k * ch, slot)
            o_emitted[0] += 1

        def process_x(k):
            sl = pl.ds(k * ch, ch)
            rx = pltpu.make_async_remote_copy(
                src_ref=recv_x.at[sl, :],
                dst_ref=recv_x.at[sl, :],
                send_sem=x_send.at[k],
                recv_sem=x_recv.at[k],
                device_id=xnbr,
                device_id_type=pl.DeviceIdType.MESH,
            )
            rx.wait_recv()
            xin_cps[k].wait()
            slot = o_emitted[0] % 2
            if len(o_cps) >= 2:
                o_cps[-2].wait()
            o_slots[slot, :, :] = (
                xin[k % 2].astype(jnp.bfloat16) + recv_x[sl, :]
            )
            emit(other_half + k * ch, slot)
            o_emitted[0] += 1
            stage_xin(k + 2)

        stage_xin(0)
        stage_xin(1)
        for k in range(_K):
            process_y(k)
            if k >= 1:
                process_x(k - 1)
        process_x(_K - 1)

        o_cps[-2].wait()
        o_cps[-1].wait()
        for k in range(_K):
            y_rdmas[k].wait_send()
            x_rdmas[k].wait_send()

    return pl.pallas_call(
        body,
        out_shape=jax.ShapeDtypeStruct((m, n), jnp.bfloat16),
        in_specs=[pl.BlockSpec(memory_space=pl.ANY)],
        out_specs=pl.BlockSpec(memory_space=pl.ANY),
        scratch_shapes=[
            pltpu.VMEM((half, n), jnp.bfloat16),
            pltpu.VMEM((half, n), jnp.bfloat16),
            pltpu.VMEM((half, n), jnp.bfloat16),
            pltpu.VMEM((2, ch, n), jnp.float32),
            pltpu.VMEM((2, ch, n), jnp.bfloat16),
            pltpu.SemaphoreType.DMA((_K,)),
            pltpu.SemaphoreType.DMA((_K,)),
            pltpu.SemaphoreType.DMA((_K,)),
            pltpu.SemaphoreType.DMA((_K,)),
            pltpu.SemaphoreType.DMA((2,)),
            pltpu.SemaphoreType.DMA((2,)),
        ],
        compiler_params=pltpu.CompilerParams(
            collective_id=0,
            vmem_limit_bytes=58 * 1024 * 1024,
        ),
    )(x)


# device time: 179203 ns/iter; 2.4015x vs baseline; 1.3028x over previous
import jax
import jax.numpy as jnp
from jax import lax
from jax.experimental import pallas as pl
from jax.experimental.pallas import tpu as pltpu

_K = 8


def kernel(x):
    m, n = x.shape
    quarter = m // 4
    ch = quarter // _K

    def body(
        x_hbm,
        out_ref,
        y_q,
        recv,
        xin,
        o_slots,
        y_send, y_recv,
        x_send, x_recv,
        z_send, z_recv,
        xin_sem, o_out,
    ):
        my_x = lax.axis_index("x")
        my_y = lax.axis_index("y")
        my_z = lax.axis_index("z")
        zp = lax.rem(my_z, 2)
        partner = (my_x, 1 - my_y, my_z)
        xnbr = (1 - my_x, my_y, my_z)
        znbr = (my_x, my_y, my_z + 1 - 2 * zp)

        q_own = (2 * my_x + zp) * quarter
        q_x = (2 * (1 - my_x) + zp) * quarter
        q_z = (2 * my_x + (1 - zp)) * quarter
        q_d = (2 * (1 - my_x) + (1 - zp)) * quarter

        barrier_sem = pltpu.get_barrier_semaphore()
        for nbr in (partner, xnbr, znbr):
            pl.semaphore_signal(
                barrier_sem, inc=1, device_id=nbr,
                device_id_type=pl.DeviceIdType.MESH,
            )

        cp = pltpu.make_async_copy(
            x_hbm.at[pl.ds(q_own, ch), :], xin.at[0], xin_sem.at[0]
        )
        cp.start()
        pl.semaphore_wait(barrier_sem, 3)

        prev = cp
        y_rdmas = []
        for k in range(_K):
            slot = k % 2
            prev.wait()
            if k + 1 < _K:
                nxt = pltpu.make_async_copy(
                    x_hbm.at[pl.ds(q_own + (k + 1) * ch, ch), :],
                    xin.at[(k + 1) % 2],
                    xin_sem.at[(k + 1) % 2],
                )
                nxt.start()
                prev = nxt
            y_q[k * ch:(k + 1) * ch, :] = xin[slot].astype(jnp.bfloat16)
            r = pltpu.make_async_remote_copy(
                src_ref=y_q.at[pl.ds(k * ch, ch), :],
                dst_ref=recv.at[pl.ds(q_own + k * ch, ch), :],
                send_sem=y_send.at[k],
                recv_sem=y_recv.at[k],
                device_id=partner,
                device_id_type=pl.DeviceIdType.MESH,
            )
            r.start()
            y_rdmas.append(r)

        events = []
        for k in range(_K):
            events.append(("y", k))
            if k >= 1:
                events.append(("x", k - 1))
                events.append(("z", k - 1))
            if k >= 2:
                events.append(("d", k - 2))
        events += [("x", _K - 1), ("z", _K - 1), ("d", _K - 2), ("d", _K - 1)]

        def ev_row(ev):
            kind, k = ev
            base = {"y": q_own, "x": q_x, "z": q_z, "d": q_d}[kind]
            return base + k * ch

        non_y = [ev for ev in events if ev[0] != "y"]
        xin_cps = {}

        def stage(p):
            if p < len(non_y):
                xcp = pltpu.make_async_copy(
                    x_hbm.at[pl.ds(ev_row(non_y[p]), ch), :],
                    xin.at[p % 2],
                    xin_sem.at[p % 2],
                )
                xcp.start()
                xin_cps[p] = xcp

        stage(0)
        stage(1)

        o_cps = []
        emitted = [0]
        x_relays = []
        z_relays = []

        def emit(row_start, a_vec):
            slot = emitted[0] % 2
            if len(o_cps) >= 2:
                o_cps[-2].wait()
            o_slots[slot, :, :] = a_vec + recv[pl.ds(row_start, ch), :]
            ocp = pltpu.make_async_copy(
                o_slots.at[slot],
                out_ref.at[pl.ds(row_start, ch), :],
                o_out.at[slot],
            )
            ocp.start()
            o_cps.append(ocp)
            emitted[0] += 1

        def relay(row_start, target, sems_send, sems_recv, idx, bucket):
            r = pltpu.make_async_remote_copy(
                src_ref=recv.at[pl.ds(row_start, ch), :],
                dst_ref=recv.at[pl.ds(row_start, ch), :],
                send_sem=sems_send.at[idx],
                recv_sem=sems_recv.at[idx],
                device_id=target,
                device_id_type=pl.DeviceIdType.MESH,
            )
            r.start()
            bucket.append(r)

        def wait_recv_only(row_start, sems_recv, idx):
            rx = pltpu.make_async_remote_copy(
                src_ref=recv.at[pl.ds(row_start, ch), :],
                dst_ref=recv.at[pl.ds(row_start, ch), :],
                send_sem=y_send.at[0],
                recv_sem=sems_recv.at[idx],
                device_id=partner,
                device_id_type=pl.DeviceIdType.MESH,
            )
            rx.wait_recv()

        p = 0
        for ev in events:
            kind, k = ev
            row = ev_row(ev)
            if kind == "y":
                y_rdmas[k].wait_recv()
                relay(row, xnbr, x_send, x_recv, k, x_relays)
                relay(row, znbr, z_send, z_recv, k, z_relays)
                emit(row, y_q[pl.ds(k * ch, ch), :])
            else:
                if kind == "x":
                    wait_recv_only(row, x_recv, k)
                    if k % 2 == 1:
                        relay(row, znbr, z_send, z_recv, _K + k // 2, z_relays)
                elif kind == "z":
                    wait_recv_only(row, z_recv, k)
                    if k % 2 == 0:
                        relay(row, xnbr, x_send, x_recv, _K + k // 2, x_relays)
                else:
                    if k % 2 == 0:
                        wait_recv_only(row, x_recv, _K + k // 2)
                    else:
                        wait_recv_only(row, z_recv, _K + k // 2)
                xin_cps[p].wait()
                emit(row, xin[p % 2].astype(jnp.bfloat16))
                stage(p + 2)
                p += 1

        o_cps[-2].wait()
        o_cps[-1].wait()
        for r in y_rdmas:
            r.wait_send()
        for r in x_relays:
            r.wait_send()
        for r in z_relays:
            r.wait_send()

    return pl.pallas_call(
        body,
        out_shape=jax.ShapeDtypeStruct((m, n), jnp.bfloat16),
        in_specs=[pl.BlockSpec(memory_space=pl.ANY)],
        out_specs=pl.BlockSpec(memory_space=pl.ANY),
        scratch_shapes=[
            pltpu.VMEM((quarter, n), jnp.bfloat16),
            pltpu.VMEM((m, n), jnp.bfloat16),
            pltpu.VMEM((2, ch, n), jnp.float32),
            pltpu.VMEM((2, ch, n), jnp.bfloat16),
            pltpu.SemaphoreType.DMA((_K,)),
            pltpu.SemaphoreType.DMA((_K,)),
            pltpu.SemaphoreType.DMA((_K + _K // 2,)),
            pltpu.SemaphoreType.DMA((_K + _K // 2,)),
            pltpu.SemaphoreType.DMA((_K + _K // 2,)),
            pltpu.SemaphoreType.DMA((_K + _K // 2,)),
            pltpu.SemaphoreType.DMA((2,)),
            pltpu.SemaphoreType.DMA((2,)),
        ],
        compiler_params=pltpu.CompilerParams(
            collective_id=0,
            vmem_limit_bytes=56 * 1024 * 1024,
        ),
    )(x)
